# Initial kernel scaffold; baseline (speedup 1.0000x reference)
import jax
import jax.numpy as jnp
from jax.experimental import pallas as pl


def kernel(x, edge_index, batch):
    raise NotImplementedError("write your pallas kernel here")


# speedup vs baseline: 5.1347x; 5.1347x over previous
import functools

import jax
import jax.numpy as jnp
from jax import lax
from jax.experimental import pallas as pl
from jax.experimental.pallas import tpu as pltpu
from jax.experimental.pallas import tpu_sc as plsc

_N = 50000
_C = 192
_E = 800000
_B = 128
_DEG = 64
_AVG = 9.43

_NC = 2
_NS = 16
_NW = _NC * _NS
_L = 16
_SEG_PER = _B // _NW
_NCG = _C // _L
_K = 128
_QB = 3136
_QE = 25024


def _lane_sum(ref, row_chunks, chunk):
    acc = ref[0, pl.ds(chunk * _L, _L)]
    for r in range(1, _L):
        acc = acc + ref[r, pl.ds(chunk * _L, _L)]
    return acc


def _sc_body(x_hbm, batch_hbm, edge_hbm,
             sum_hbm, sq_hbm, mx_hbm, mn_hbm, cnt_hbm, eh_hbm,
             bbuf, ebuf, bh16, eh16, call, eall, cvec, evec, rowbuf,
             sumb, sqb, mxb, mnb, spb, spe):
    c = lax.axis_index("c")
    s = lax.axis_index("s")
    w = c * _NS + s
    lanes = lax.iota(jnp.int32, _L)
    ones_i = jnp.ones((_L,), jnp.int32)
    zeros_i = jnp.zeros((_L,), jnp.int32)

    for r in range(_L):
        for k in range(_B // _L):
            bh16[r, pl.ds(k * _L, _L)] = zeros_i
        for k in range(_DEG // _L):
            eh16[r, pl.ds(k * _L, _L)] = zeros_i

    bstart = s * _QB
    bclamp = jnp.minimum(bstart, _N - _QB)
    bbase = bstart - bclamp
    blen = jnp.minimum(_QB, _N - bstart)
    pltpu.sync_copy(batch_hbm.at[pl.ds(bclamp, _QB)], bbuf)

    def bh_step(g, carry):
        idx = bbuf[pl.ds(bbase + g * _L, _L)]
        plsc.addupdate_scatter(bh16, [lanes, idx], ones_i)
        return carry
    lax.fori_loop(0, blen // _L, bh_step, 0)

    estart = w * _QE
    eclamp = jnp.minimum(estart, _E - _QE)
    ebase = estart - eclamp
    elen = jnp.maximum(jnp.minimum(_QE, _E - estart), 0)
    pltpu.sync_copy(edge_hbm.at[pl.ds(eclamp, _QE)], ebuf)

    def eh_step(g, carry):
        idx = ebuf[pl.ds(ebase + g * _L, _L)]
        plsc.addupdate_scatter(eh16, [lanes, idx], ones_i)
        return carry
    lax.fori_loop(0, elen // _L, eh_step, 0)

    for k in range(_B // _L):
        cvec[pl.ds(k * _L, _L)] = _lane_sum(bh16, _B // _L, k)
    for k in range(_DEG // _L):
        evec[pl.ds(k * _L, _L)] = _lane_sum(eh16, _DEG // _L, k)
    pltpu.sync_copy(cvec, spb.at[s])
    pltpu.sync_copy(evec, spe.at[s])
    plsc.subcore_barrier()

    pltpu.sync_copy(spb, call)
    for k in range(_B // _L):
        cvec[pl.ds(k * _L, _L)] = _lane_sum(call, _B // _L, k)

    @pl.when(s == 0)
    def _():
        pltpu.sync_copy(spe, eall)
        for k in range(_DEG // _L):
            evec[pl.ds(k * _L, _L)] = _lane_sum(eall, _DEG // _L, k)
        pltpu.sync_copy(evec, eh_hbm.at[c])
        pltpu.sync_copy(cvec, cnt_hbm.at[c])

    b0 = w * _SEG_PER
    k0 = b0 // _L
    jb = b0 % _L

    def pfx_step(jj, p):
        return p + jnp.sum(cvec[pl.ds(jj * _L, _L)])
    prefix = lax.fori_loop(0, k0, pfx_step, jnp.int32(0))
    ck = cvec[pl.ds(k0 * _L, _L)]
    excl = plsc.cumsum(ck) - ck

    f0 = jnp.zeros((_L,), jnp.float32)
    fneg = jnp.full((_L,), -jnp.inf, jnp.float32)
    fpos = jnp.full((_L,), jnp.inf, jnp.float32)

    for j in range(_SEG_PER):
        boff = jb + j
        st = prefix + jnp.sum(jnp.where(lanes == boff, excl, 0))
        cn = jnp.sum(jnp.where(lanes == boff, ck, 0))
        nch = (cn + _K - 1) // _K

        def chunk_body(i, accs):
            a0 = st + i * _K
            cs = jnp.minimum(a0, _N - _K)
            base = a0 - cs
            rem = jnp.minimum(_K, cn - i * _K)
            pltpu.sync_copy(x_hbm.at[pl.ds(cs, _K)], rowbuf)

            def row_body(r, accs):
                sums, sqs, mxs, mns = accs
                ns, nq, nx, nn = [], [], [], []
                for g in range(_NCG):
                    v = rowbuf[r, pl.ds(g * _L, _L)]
                    ns.append(sums[g] + v)
                    nq.append(sqs[g] + v * v)
                    nx.append(jnp.maximum(mxs[g], v))
                    nn.append(jnp.minimum(mns[g], v))
                return (tuple(ns), tuple(nq), tuple(nx), tuple(nn))
            return lax.fori_loop(base, base + rem, row_body, accs)

        init = (tuple(f0 for _ in range(_NCG)),
                tuple(f0 for _ in range(_NCG)),
                tuple(fneg for _ in range(_NCG)),
                tuple(fpos for _ in range(_NCG)))
        sums, sqs, mxs, mns = lax.fori_loop(0, nch, chunk_body, init)
        for g in range(_NCG):
            sumb[j, pl.ds(g * _L, _L)] = sums[g]
            sqb[j, pl.ds(g * _L, _L)] = sqs[g]
            mxb[j, pl.ds(g * _L, _L)] = mxs[g]
            mnb[j, pl.ds(g * _L, _L)] = mns[g]

    pltpu.sync_copy(sumb, sum_hbm.at[pl.ds(b0, _SEG_PER)])
    pltpu.sync_copy(sqb, sq_hbm.at[pl.ds(b0, _SEG_PER)])
    pltpu.sync_copy(mxb, mx_hbm.at[pl.ds(b0, _SEG_PER)])
    pltpu.sync_copy(mnb, mn_hbm.at[pl.ds(b0, _SEG_PER)])


@functools.lru_cache(maxsize=1)
def _build_sc_stats():
  return functools.partial(
    pl.kernel,
    out_type=[
        jax.ShapeDtypeStruct((_B, _C), jnp.float32),
        jax.ShapeDtypeStruct((_B, _C), jnp.float32),
        jax.ShapeDtypeStruct((_B, _C), jnp.float32),
        jax.ShapeDtypeStruct((_B, _C), jnp.float32),
        jax.ShapeDtypeStruct((_NC, _B), jnp.int32),
        jax.ShapeDtypeStruct((_NC, _DEG), jnp.int32),
    ],
    mesh=plsc.VectorSubcoreMesh(core_axis_name="c", subcore_axis_name="s",
                                num_cores=_NC, num_subcores=_NS),
    compiler_params=pltpu.CompilerParams(use_tc_tiling_on_sc=False,
                                         needs_layout_passes=False),
    scratch_types=[
        pltpu.VMEM((_QB,), jnp.int32),
        pltpu.VMEM((_QE,), jnp.int32),
        pltpu.VMEM((_L, _B), jnp.int32),
        pltpu.VMEM((_L, _DEG), jnp.int32),
        pltpu.VMEM((_L, _B), jnp.int32),
        pltpu.VMEM((_L, _DEG), jnp.int32),
        pltpu.VMEM((_B,), jnp.int32),
        pltpu.VMEM((_DEG,), jnp.int32),
        pltpu.VMEM((_K, _C), jnp.float32),
        pltpu.VMEM((_SEG_PER, _C), jnp.float32),
        pltpu.VMEM((_SEG_PER, _C), jnp.float32),
        pltpu.VMEM((_SEG_PER, _C), jnp.float32),
        pltpu.VMEM((_SEG_PER, _C), jnp.float32),
        pltpu.VMEM_SHARED((_NS, _B), jnp.int32),
        pltpu.VMEM_SHARED((_NS, _DEG), jnp.int32),
    ],
  )(_sc_body)


def _fin_body(cntc_ref, eh_ref, sum_ref, sq_ref, mx_ref, mn_ref, out_ref):
    d = (eh_ref[0:1, :] + eh_ref[1:2, :]).astype(jnp.float32)
    l1 = jnp.log(d + 1.0) / _AVG
    deg = jnp.concatenate([jnp.ones_like(l1), l1, 1.0 / l1], axis=1)

    count = cntc_ref[...]
    count_c = jnp.clip(count, 1.0, None)
    mean = sum_ref[...] / count_c
    var_sum = jnp.maximum(sq_ref[...] - sum_ref[...] * mean, 0.0)
    denom = jnp.clip(count - 1.0, 1.0, None) + 1e-6
    std = jnp.sqrt(var_sum / denom)

    out_ref[0] = mean * deg
    out_ref[1] = std * deg
    out_ref[2] = mx_ref[...] * deg
    out_ref[3] = mn_ref[...] * deg


_finalize = pl.pallas_call(
    _fin_body,
    out_shape=jax.ShapeDtypeStruct((4, _B, _C), jnp.float32),
)


def kernel(x, edge_index, batch):
    e1 = edge_index[1].astype(jnp.int32)
    seg_sum, seg_sq, seg_mx, seg_mn, cnt2, eh2 = _build_sc_stats()(
        x, batch.astype(jnp.int32), e1)
    cnt_col = cnt2[0].astype(jnp.float32).reshape(_B, 1)
    out4 = _finalize(cnt_col, eh2, seg_sum, seg_sq, seg_mx, seg_mn)
    return jnp.transpose(out4, (1, 0, 2)).reshape(_B, 4 * _C)
